# baseline (speedup 1.0000x reference)
import functools

import jax
import jax.numpy as jnp
from jax import lax
from jax.experimental import pallas as pl
from jax.experimental.pallas import tpu as pltpu
from jax.experimental.pallas import tpu_sc as plsc

TOPK = 2
BT = 256
NW = 32
LANES = 16


def _dispatch(x_flat, src):
    t, h = x_flat.shape
    p = src.shape[0]
    rows_per_w = p // NW
    c = 64
    nchunks = rows_per_w // c
    mesh = plsc.VectorSubcoreMesh(core_axis_name="c", subcore_axis_name="s")

    @functools.partial(
        pl.kernel,
        out_type=jax.ShapeDtypeStruct((p, h), jnp.float32),
        mesh=mesh,
        scratch_types=[
            pltpu.VMEM((c,), jnp.int32),
            pltpu.VMEM((c, h), jnp.float32),
            pltpu.SemaphoreType.DMA,
        ],
    )
    def k(x_hbm, src_hbm, out_hbm, idx_v, rows_v, sem):
        wid = lax.axis_index("s") * 2 + lax.axis_index("c")
        base = wid * rows_per_w

        def body(ci, carry):
            off = base + ci * c
            pltpu.sync_copy(src_hbm.at[pl.ds(off, c)], idx_v)
            pltpu.async_copy(x_hbm.at[idx_v], rows_v, sem).wait()
            pltpu.sync_copy(rows_v, out_hbm.at[pl.ds(off, c)])
            return carry

        lax.fori_loop(0, nchunks, body, 0)

    return k(x_flat, src)


def _combine(ys, pos2):
    p, h = ys.shape
    t = pos2.shape[0] // TOPK
    tok_per_w = t // NW
    ct = 16
    nchunks = tok_per_w // ct
    mesh = plsc.VectorSubcoreMesh(core_axis_name="c", subcore_axis_name="s")

    @functools.partial(
        pl.kernel,
        out_type=jax.ShapeDtypeStruct((t, h), jnp.float32),
        mesh=mesh,
        scratch_types=[
            pltpu.VMEM((TOPK * ct,), jnp.int32),
            pltpu.VMEM((TOPK * ct, h), jnp.float32),
            pltpu.VMEM((ct, h), jnp.float32),
            pltpu.SemaphoreType.DMA,
        ],
    )
    def k(ys_hbm, pos_hbm, out_hbm, idx_v, rows_v, out_v, sem):
        wid = lax.axis_index("s") * 2 + lax.axis_index("c")
        tbase = wid * tok_per_w

        def chunk(ci, carry):
            toff = tbase + ci * ct
            pltpu.sync_copy(pos_hbm.at[pl.ds(TOPK * toff, TOPK * ct)], idx_v)
            pltpu.async_copy(ys_hbm.at[idx_v], rows_v, sem).wait()

            def tokbody(i, carry2):
                def hbody(j, carry3):
                    sl = pl.ds(j * LANES, LANES)
                    out_v[i, sl] = rows_v[2 * i, sl] + rows_v[2 * i + 1, sl]
                    return carry3

                lax.fori_loop(0, h // LANES, hbody, 0)
                return carry2

            lax.fori_loop(0, ct, tokbody, 0)
            pltpu.sync_copy(out_v, out_hbm.at[pl.ds(toff, ct)])
            return carry

        lax.fori_loop(0, nchunks, chunk, 0)

    return k(ys, pos2)


def _gating_body(x_ref, wg_ref, logits_ref, topi_ref, topw_ref):
    lg = jnp.dot(x_ref[...], wg_ref[...], preferred_element_type=jnp.float32)
    logits_ref[...] = lg
    e = lg.shape[-1]
    col = jax.lax.broadcasted_iota(jnp.int32, lg.shape, 1)
    i1 = jnp.argmax(lg, axis=-1).astype(jnp.int32)
    m1 = jnp.max(lg, axis=-1)
    masked = jnp.where(col == i1[:, None], -jnp.inf, lg)
    i2 = jnp.argmax(masked, axis=-1).astype(jnp.int32)
    m2 = jnp.max(masked, axis=-1)
    a = jnp.exp(m2 - m1)
    w1 = 1.0 / (1.0 + a)
    topi_ref[...] = jnp.stack([i1, i2], axis=-1)
    topw_ref[...] = jnp.stack([w1, 1.0 - w1], axis=-1)


def _gating(x_flat, Wgate):
    t, h = x_flat.shape
    e = Wgate.shape[1]
    tg = 1024
    return pl.pallas_call(
        _gating_body,
        grid=(t // tg,),
        in_specs=[
            pl.BlockSpec((tg, h), lambda i: (i, 0)),
            pl.BlockSpec((h, e), lambda i: (0, 0)),
        ],
        out_specs=[
            pl.BlockSpec((tg, e), lambda i: (i, 0)),
            pl.BlockSpec((tg, TOPK), lambda i: (i, 0)),
            pl.BlockSpec((tg, TOPK), lambda i: (i, 0)),
        ],
        out_shape=[
            jax.ShapeDtypeStruct((t, e), jnp.float32),
            jax.ShapeDtypeStruct((t, TOPK), jnp.int32),
            jax.ShapeDtypeStruct((t, TOPK), jnp.float32),
        ],
    )(x_flat, Wgate)


def _ffn_body(be_ref, xs_ref, wrow_ref, wg_ref, wu_ref, wd_ref, ys_ref):
    del be_ref
    xb = xs_ref[...].astype(jnp.bfloat16)
    g = jnp.dot(xb, wg_ref[0], preferred_element_type=jnp.float32)
    u = jnp.dot(xb, wu_ref[0], preferred_element_type=jnp.float32)
    h1 = (g * jax.nn.sigmoid(g) * u).astype(jnp.bfloat16)
    o = jnp.dot(h1, wd_ref[0], preferred_element_type=jnp.float32)
    ys_ref[...] = o * wrow_ref[0, 0, :][:, None]


def _grouped_ffn(xs, wrow3d, Wg, Wu, Wd, block_expert):
    p, h = xs.shape
    _, _, f = Wg.shape
    nb = p // BT
    grid_spec = pltpu.PrefetchScalarGridSpec(
        num_scalar_prefetch=1,
        grid=(nb,),
        in_specs=[
            pl.BlockSpec((BT, h), lambda i, be: (i, 0)),
            pl.BlockSpec((1, 1, BT), lambda i, be: (i, 0, 0)),
            pl.BlockSpec((1, h, f), lambda i, be: (be[i], 0, 0)),
            pl.BlockSpec((1, h, f), lambda i, be: (be[i], 0, 0)),
            pl.BlockSpec((1, f, h), lambda i, be: (be[i], 0, 0)),
        ],
        out_specs=pl.BlockSpec((BT, h), lambda i, be: (i, 0)),
    )
    return pl.pallas_call(
        _ffn_body,
        grid_spec=grid_spec,
        out_shape=jax.ShapeDtypeStruct((p, h), jnp.float32),
    )(block_expert, xs, wrow3d, Wg, Wu, Wd)


@jax.jit
def kernel(x, Wgate, Wg, Wu, Wd):
    b, s, h = x.shape
    e = Wgate.shape[1]
    t = b * s
    a = t * TOPK
    p = a + e * BT
    nb = p // BT

    x_flat = x.reshape(t, h)
    logits, topi, topw = _gating(x_flat, Wgate)

    ef = topi.reshape(-1)
    oh = jax.nn.one_hot(ef, e, dtype=jnp.int32)
    cnt = oh.sum(axis=0)
    rank = jnp.take_along_axis(jnp.cumsum(oh, axis=0) - oh, ef[:, None], axis=1)[:, 0]
    cnt_pad = ((cnt + BT - 1) // BT) * BT
    ends = jnp.cumsum(cnt_pad)
    aligned_off = ends - cnt_pad
    slot = aligned_off[ef] + rank
    tok = jnp.arange(a, dtype=jnp.int32) // TOPK
    src = jnp.zeros((p,), jnp.int32).at[slot].set(tok)
    wrow = jnp.zeros((p,), jnp.float32).at[slot].set(topw.reshape(-1))
    blockstart = jnp.arange(nb, dtype=jnp.int32) * BT
    block_expert = jnp.minimum(
        jnp.sum((blockstart[:, None] >= ends[None, :]).astype(jnp.int32), axis=1),
        e - 1,
    ).astype(jnp.int32)

    xs = _dispatch(x_flat, src)

    ys = _grouped_ffn(
        xs,
        wrow.reshape(nb, 1, BT),
        Wg.astype(jnp.bfloat16),
        Wu.astype(jnp.bfloat16),
        Wd.astype(jnp.bfloat16),
        block_expert,
    )

    out = _combine(ys, slot)

    return out.reshape(b, s, h), logits


# speedup vs baseline: 1.3639x; 1.3639x over previous
import functools

import jax
import jax.numpy as jnp
from jax import lax
from jax.experimental import pallas as pl
from jax.experimental.pallas import tpu as pltpu
from jax.experimental.pallas import tpu_sc as plsc

TOPK = 2
BT = 256
NW = 32
LANES = 16


def _gating_body(x_ref, wg_ref, logits_ref, topi_ref, topw_ref):
    lg = jnp.dot(x_ref[...], wg_ref[...], preferred_element_type=jnp.float32)
    logits_ref[...] = lg
    col = jax.lax.broadcasted_iota(jnp.int32, lg.shape, 1)
    i1 = jnp.argmax(lg, axis=-1).astype(jnp.int32)
    m1 = jnp.max(lg, axis=-1)
    masked = jnp.where(col == i1[:, None], -jnp.inf, lg)
    i2 = jnp.argmax(masked, axis=-1).astype(jnp.int32)
    m2 = jnp.max(masked, axis=-1)
    a = jnp.exp(m2 - m1)
    w1 = 1.0 / (1.0 + a)
    topi_ref[...] = jnp.stack([i1, i2], axis=-1)
    topw_ref[...] = jnp.stack([w1, 1.0 - w1], axis=-1)


def _gating(x_flat, Wgate):
    t, h = x_flat.shape
    e = Wgate.shape[1]
    tg = 1024
    return pl.pallas_call(
        _gating_body,
        grid=(t // tg,),
        in_specs=[
            pl.BlockSpec((tg, h), lambda i: (i, 0)),
            pl.BlockSpec((h, e), lambda i: (0, 0)),
        ],
        out_specs=[
            pl.BlockSpec((tg, e), lambda i: (i, 0)),
            pl.BlockSpec((tg, TOPK), lambda i: (i, 0)),
            pl.BlockSpec((tg, TOPK), lambda i: (i, 0)),
        ],
        out_shape=[
            jax.ShapeDtypeStruct((t, e), jnp.float32),
            jax.ShapeDtypeStruct((t, TOPK), jnp.int32),
            jax.ShapeDtypeStruct((t, TOPK), jnp.float32),
        ],
    )(x_flat, Wgate)


def _dispatch(x_flat, slot0, slot1, p):
    t, h = x_flat.shape
    tok_per_w = t // NW
    ct = 64
    nchunks = tok_per_w // ct
    mesh = plsc.VectorSubcoreMesh(core_axis_name="c", subcore_axis_name="s")

    @functools.partial(
        pl.kernel,
        out_type=jax.ShapeDtypeStruct((p, h), jnp.float32),
        mesh=mesh,
        scratch_types=[
            pltpu.VMEM((ct,), jnp.int32),
            pltpu.VMEM((ct,), jnp.int32),
            pltpu.VMEM((ct, h), jnp.float32),
            pltpu.SemaphoreType.DMA,
        ],
    )
    def k(x_hbm, s0_hbm, s1_hbm, out_hbm, s0_v, s1_v, rows_v, sem):
        wid = lax.axis_index("s") * 2 + lax.axis_index("c")
        base = wid * tok_per_w

        def body(ci, carry):
            toff = base + ci * ct
            pltpu.sync_copy(x_hbm.at[pl.ds(toff, ct)], rows_v)
            pltpu.sync_copy(s0_hbm.at[pl.ds(toff, ct)], s0_v)
            pltpu.sync_copy(s1_hbm.at[pl.ds(toff, ct)], s1_v)
            cp0 = pltpu.async_copy(rows_v, out_hbm.at[s0_v], sem)
            cp1 = pltpu.async_copy(rows_v, out_hbm.at[s1_v], sem)
            cp0.wait()
            cp1.wait()
            return carry

        lax.fori_loop(0, nchunks, body, 0)

    return k(x_flat, slot0, slot1)


def _combine(ys, slot_flat, w_flat):
    p, h = ys.shape
    t = slot_flat.shape[0] // TOPK
    tok_per_w = t // NW
    ct = 8
    nchunks = tok_per_w // ct
    mesh = plsc.VectorSubcoreMesh(core_axis_name="c", subcore_axis_name="s")

    @functools.partial(
        pl.kernel,
        out_type=jax.ShapeDtypeStruct((t, h), jnp.float32),
        mesh=mesh,
        scratch_types=[
            pltpu.VMEM((TOPK * ct,), jnp.int32),
            pltpu.VMEM((TOPK * ct,), jnp.float32),
            pltpu.VMEM((TOPK * ct, h), jnp.float32),
            pltpu.VMEM((ct, h), jnp.float32),
            pltpu.SemaphoreType.DMA,
        ],
    )
    def k(ys_hbm, pos_hbm, w_hbm, out_hbm, idx_v, w_v, rows_v, out_v, sem):
        wid = lax.axis_index("s") * 2 + lax.axis_index("c")
        tbase = wid * tok_per_w

        def chunk(ci, carry):
            toff = tbase + ci * ct
            pltpu.sync_copy(pos_hbm.at[pl.ds(TOPK * toff, TOPK * ct)], idx_v)
            pltpu.sync_copy(w_hbm.at[pl.ds(TOPK * toff, TOPK * ct)], w_v)
            pltpu.async_copy(ys_hbm.at[idx_v], rows_v, sem).wait()
            wv = w_v[...]

            for i in range(ct):
                w0 = wv[2 * i]
                w1 = wv[2 * i + 1]

                def hbody(j, carry3, i=i, w0=w0, w1=w1):
                    sl = pl.ds(j * LANES, LANES)
                    out_v[i, sl] = w0 * rows_v[2 * i, sl] + w1 * rows_v[2 * i + 1, sl]
                    return carry3

                lax.fori_loop(0, h // LANES, hbody, 0)

            pltpu.sync_copy(out_v, out_hbm.at[pl.ds(toff, ct)])
            return carry

        lax.fori_loop(0, nchunks, chunk, 0)

    return k(ys, slot_flat, w_flat)


def _ffn_body(be_ref, xs_ref, wg_ref, wu_ref, wd_ref, ys_ref):
    del be_ref
    xb = xs_ref[...].astype(jnp.bfloat16)
    g = jnp.dot(xb, wg_ref[0], preferred_element_type=jnp.float32)
    u = jnp.dot(xb, wu_ref[0], preferred_element_type=jnp.float32)
    h1 = (g * jax.nn.sigmoid(g) * u).astype(jnp.bfloat16)
    ys_ref[...] = jnp.dot(h1, wd_ref[0], preferred_element_type=jnp.float32)


def _grouped_ffn(xs, Wg, Wu, Wd, block_expert):
    p, h = xs.shape
    _, _, f = Wg.shape
    nb = p // BT
    grid_spec = pltpu.PrefetchScalarGridSpec(
        num_scalar_prefetch=1,
        grid=(nb,),
        in_specs=[
            pl.BlockSpec((BT, h), lambda i, be: (i, 0)),
            pl.BlockSpec((1, h, f), lambda i, be: (be[i], 0, 0)),
            pl.BlockSpec((1, h, f), lambda i, be: (be[i], 0, 0)),
            pl.BlockSpec((1, f, h), lambda i, be: (be[i], 0, 0)),
        ],
        out_specs=pl.BlockSpec((BT, h), lambda i, be: (i, 0)),
    )
    return pl.pallas_call(
        _ffn_body,
        grid_spec=grid_spec,
        out_shape=jax.ShapeDtypeStruct((p, h), jnp.float32),
    )(block_expert, xs, Wg, Wu, Wd)


@jax.jit
def kernel(x, Wgate, Wg, Wu, Wd):
    b, s, h = x.shape
    e = Wgate.shape[1]
    t = b * s
    a = t * TOPK
    p = a + e * BT
    nb = p // BT

    x_flat = x.reshape(t, h)
    logits, topi, topw = _gating(x_flat, Wgate)

    ef = topi.reshape(-1)
    oh = jax.nn.one_hot(ef, e, dtype=jnp.int32)
    cnt = oh.sum(axis=0)
    rank = jnp.sum((jnp.cumsum(oh, axis=0) - oh) * oh, axis=1)
    cnt_pad = ((cnt + BT - 1) // BT) * BT
    ends = jnp.cumsum(cnt_pad)
    aligned_off = ends - cnt_pad
    slot = aligned_off[ef] + rank
    slot2 = slot.reshape(t, TOPK)
    blockstart = jnp.arange(nb, dtype=jnp.int32) * BT
    block_expert = jnp.minimum(
        jnp.sum((blockstart[:, None] >= ends[None, :]).astype(jnp.int32), axis=1),
        e - 1,
    ).astype(jnp.int32)

    xs = _dispatch(x_flat, slot2[:, 0], slot2[:, 1], p)

    ys = _grouped_ffn(
        xs,
        Wg.astype(jnp.bfloat16),
        Wu.astype(jnp.bfloat16),
        Wd.astype(jnp.bfloat16),
        block_expert,
    )

    out = _combine(ys, slot, topw.reshape(-1))

    return out.reshape(b, s, h), logits


# speedup vs baseline: 1.4628x; 1.0725x over previous
import functools

import jax
import jax.numpy as jnp
from jax import lax
from jax.experimental import pallas as pl
from jax.experimental.pallas import tpu as pltpu
from jax.experimental.pallas import tpu_sc as plsc

TOPK = 2
BT = 256
NW = 32
LANES = 16


def _gating_body(x_ref, wg_ref, logits_ref, topi_ref, topw_ref):
    lg = jnp.dot(x_ref[...], wg_ref[...], preferred_element_type=jnp.float32)
    logits_ref[...] = lg
    col = jax.lax.broadcasted_iota(jnp.int32, lg.shape, 1)
    i1 = jnp.argmax(lg, axis=-1).astype(jnp.int32)
    m1 = jnp.max(lg, axis=-1)
    masked = jnp.where(col == i1[:, None], -jnp.inf, lg)
    i2 = jnp.argmax(masked, axis=-1).astype(jnp.int32)
    m2 = jnp.max(masked, axis=-1)
    a = jnp.exp(m2 - m1)
    w1 = 1.0 / (1.0 + a)
    topi_ref[...] = jnp.stack([i1, i2], axis=-1)
    topw_ref[...] = jnp.stack([w1, 1.0 - w1], axis=-1)


def _gating(x_flat, Wgate):
    t, h = x_flat.shape
    e = Wgate.shape[1]
    tg = 1024
    return pl.pallas_call(
        _gating_body,
        grid=(t // tg,),
        in_specs=[
            pl.BlockSpec((tg, h), lambda i: (i, 0)),
            pl.BlockSpec((h, e), lambda i: (0, 0)),
        ],
        out_specs=[
            pl.BlockSpec((tg, e), lambda i: (i, 0)),
            pl.BlockSpec((tg, TOPK), lambda i: (i, 0)),
            pl.BlockSpec((tg, TOPK), lambda i: (i, 0)),
        ],
        out_shape=[
            jax.ShapeDtypeStruct((t, e), jnp.float32),
            jax.ShapeDtypeStruct((t, TOPK), jnp.int32),
            jax.ShapeDtypeStruct((t, TOPK), jnp.float32),
        ],
    )(x_flat, Wgate)


def _dispatch(x_flat, slot0, slot1, p):
    t, h = x_flat.shape
    tok_per_w = t // NW
    ct = 64
    nchunks = tok_per_w // ct
    mesh = plsc.VectorSubcoreMesh(core_axis_name="c", subcore_axis_name="s")

    @functools.partial(
        pl.kernel,
        out_type=jax.ShapeDtypeStruct((p, h), jnp.float32),
        mesh=mesh,
        scratch_types=[
            pltpu.VMEM((ct,), jnp.int32),
            pltpu.VMEM((ct,), jnp.int32),
            pltpu.VMEM((ct, h), jnp.float32),
            pltpu.SemaphoreType.DMA,
        ],
    )
    def k(x_hbm, s0_hbm, s1_hbm, out_hbm, s0_v, s1_v, rows_v, sem):
        wid = lax.axis_index("s") * 2 + lax.axis_index("c")
        base = wid * tok_per_w

        def body(ci, carry):
            toff = base + ci * ct
            pltpu.sync_copy(x_hbm.at[pl.ds(toff, ct)], rows_v)
            pltpu.sync_copy(s0_hbm.at[pl.ds(toff, ct)], s0_v)
            pltpu.sync_copy(s1_hbm.at[pl.ds(toff, ct)], s1_v)
            cp0 = pltpu.async_copy(rows_v, out_hbm.at[s0_v], sem)
            cp1 = pltpu.async_copy(rows_v, out_hbm.at[s1_v], sem)
            cp0.wait()
            cp1.wait()
            return carry

        lax.fori_loop(0, nchunks, body, 0)

    return k(x_flat, slot0, slot1)


def _combine(ys, slot_flat, w_flat):
    p, h = ys.shape
    t = slot_flat.shape[0] // TOPK
    tok_per_w = t // NW
    ct = 16
    nchunks = tok_per_w // ct
    mesh = plsc.VectorSubcoreMesh(core_axis_name="c", subcore_axis_name="s")

    @functools.partial(
        pl.kernel,
        out_type=jax.ShapeDtypeStruct((t, h), jnp.float32),
        mesh=mesh,
        scratch_types=[
            pltpu.VMEM((tok_per_w * TOPK,), jnp.int32),
            pltpu.VMEM((tok_per_w * TOPK,), jnp.float32),
            pltpu.VMEM((2, TOPK * ct, h), jnp.float32),
            pltpu.VMEM((ct, h), jnp.float32),
            pltpu.SemaphoreType.DMA,
            pltpu.SemaphoreType.DMA,
        ],
    )
    def k(ys_hbm, pos_hbm, w_hbm, out_hbm, idx_v, w_v, rows_v, out_v, sem0, sem1):
        wid = lax.axis_index("s") * 2 + lax.axis_index("c")
        tbase = wid * tok_per_w
        sems = [sem0, sem1]
        pltpu.sync_copy(pos_hbm.at[pl.ds(TOPK * tbase, TOPK * tok_per_w)], idx_v)
        pltpu.sync_copy(w_hbm.at[pl.ds(TOPK * tbase, TOPK * tok_per_w)], w_v)

        def gather(g, b):
            return pltpu.async_copy(
                ys_hbm.at[idx_v.at[pl.ds(g * TOPK * ct, TOPK * ct)]],
                rows_v.at[b],
                sems[b],
            )

        gather(0, 0)
        for g in range(nchunks):
            b = g % 2
            if g + 1 < nchunks:
                gather(g + 1, 1 - b)
            pltpu.make_async_copy(
                ys_hbm.at[idx_v.at[pl.ds(g * TOPK * ct, TOPK * ct)]],
                rows_v.at[b],
                sems[b],
            ).wait()
            wva = w_v[pl.ds(g * TOPK * ct, LANES)]
            wvb = w_v[pl.ds(g * TOPK * ct + LANES, LANES)]
            for i in range(ct):
                wv = wva if 2 * i < LANES else wvb
                w0 = wv[(2 * i) % LANES]
                w1 = wv[(2 * i + 1) % LANES]

                def hbody(j, carry3, b=b, i=i, w0=w0, w1=w1):
                    sl = pl.ds(j * LANES, LANES)
                    out_v[i, sl] = (
                        w0 * rows_v[b, 2 * i, sl] + w1 * rows_v[b, 2 * i + 1, sl]
                    )
                    return carry3

                lax.fori_loop(0, h // LANES, hbody, 0)

            pltpu.sync_copy(out_v, out_hbm.at[pl.ds(tbase + g * ct, ct)])

    return k(ys, slot_flat, w_flat)


def _cast_body(wg_ref, wu_ref, wd_ref, og_ref, ou_ref, od_ref):
    og_ref[...] = wg_ref[...].astype(jnp.bfloat16)
    ou_ref[...] = wu_ref[...].astype(jnp.bfloat16)
    od_ref[...] = wd_ref[...].astype(jnp.bfloat16)


def _cast_weights(Wg, Wu, Wd):
    e, h, f = Wg.shape
    nc = 2
    fc = f // nc
    return pl.pallas_call(
        _cast_body,
        grid=(e, nc),
        in_specs=[
            pl.BlockSpec((1, h, fc), lambda i, j: (i, 0, j)),
            pl.BlockSpec((1, h, fc), lambda i, j: (i, 0, j)),
            pl.BlockSpec((1, fc, h), lambda i, j: (i, j, 0)),
        ],
        out_specs=[
            pl.BlockSpec((1, h, fc), lambda i, j: (i, 0, j)),
            pl.BlockSpec((1, h, fc), lambda i, j: (i, 0, j)),
            pl.BlockSpec((1, fc, h), lambda i, j: (i, j, 0)),
        ],
        out_shape=[
            jax.ShapeDtypeStruct((e, h, f), jnp.bfloat16),
            jax.ShapeDtypeStruct((e, h, f), jnp.bfloat16),
            jax.ShapeDtypeStruct((e, f, h), jnp.bfloat16),
        ],
    )(Wg, Wu, Wd)


def _ffn_body(be_ref, xs_ref, wg_ref, wu_ref, wd_ref, ys_ref):
    del be_ref
    xb = xs_ref[...].astype(jnp.bfloat16)
    g = jnp.dot(xb, wg_ref[0], preferred_element_type=jnp.float32)
    u = jnp.dot(xb, wu_ref[0], preferred_element_type=jnp.float32)
    h1 = (g * jax.nn.sigmoid(g) * u).astype(jnp.bfloat16)
    ys_ref[...] = jnp.dot(h1, wd_ref[0], preferred_element_type=jnp.float32)


def _grouped_ffn(xs, Wg, Wu, Wd, block_expert):
    p, h = xs.shape
    _, _, f = Wg.shape
    nb = p // BT
    grid_spec = pltpu.PrefetchScalarGridSpec(
        num_scalar_prefetch=1,
        grid=(nb,),
        in_specs=[
            pl.BlockSpec((BT, h), lambda i, be: (i, 0)),
            pl.BlockSpec((1, h, f), lambda i, be: (be[i], 0, 0)),
            pl.BlockSpec((1, h, f), lambda i, be: (be[i], 0, 0)),
            pl.BlockSpec((1, f, h), lambda i, be: (be[i], 0, 0)),
        ],
        out_specs=pl.BlockSpec((BT, h), lambda i, be: (i, 0)),
    )
    return pl.pallas_call(
        _ffn_body,
        grid_spec=grid_spec,
        out_shape=jax.ShapeDtypeStruct((p, h), jnp.float32),
    )(block_expert, xs, Wg, Wu, Wd)


@jax.jit
def kernel(x, Wgate, Wg, Wu, Wd):
    b, s, h = x.shape
    e = Wgate.shape[1]
    t = b * s
    a = t * TOPK
    p = a + e * BT
    nb = p // BT

    x_flat = x.reshape(t, h)
    logits, topi, topw = _gating(x_flat, Wgate)

    ef = topi.reshape(-1)
    oh = jax.nn.one_hot(ef, e, dtype=jnp.int32)
    cnt = oh.sum(axis=0)
    rank = jnp.sum((jnp.cumsum(oh, axis=0) - oh) * oh, axis=1)
    cnt_pad = ((cnt + BT - 1) // BT) * BT
    ends = jnp.cumsum(cnt_pad)
    aligned_off = ends - cnt_pad
    slot = aligned_off[ef] + rank
    slot2 = slot.reshape(t, TOPK)
    blockstart = jnp.arange(nb, dtype=jnp.int32) * BT
    block_expert = jnp.minimum(
        jnp.sum((blockstart[:, None] >= ends[None, :]).astype(jnp.int32), axis=1),
        e - 1,
    ).astype(jnp.int32)

    wg16, wu16, wd16 = _cast_weights(Wg, Wu, Wd)
    xs = _dispatch(x_flat, slot2[:, 0], slot2[:, 1], p)

    ys = _grouped_ffn(xs, wg16, wu16, wd16, block_expert)

    out = _combine(ys, slot, topw.reshape(-1))

    return out.reshape(b, s, h), logits


# speedup vs baseline: 1.4899x; 1.0186x over previous
import functools

import jax
import jax.numpy as jnp
from jax import lax
from jax.experimental import pallas as pl
from jax.experimental.pallas import tpu as pltpu
from jax.experimental.pallas import tpu_sc as plsc

TOPK = 2
BT = 256
NW = 32
LANES = 16


SEG = 128


def _gating_body(x_ref, wg_ref, logits_ref, topi_ref, topw_ref, rank_ref,
                 cnt_ref, carry_ref):
    @pl.when(pl.program_id(0) == 0)
    def _init():
        carry_ref[...] = jnp.zeros_like(carry_ref)

    lg = jnp.dot(x_ref[...], wg_ref[...], preferred_element_type=jnp.float32)
    logits_ref[...] = lg
    tg, e = lg.shape
    col = jax.lax.broadcasted_iota(jnp.int32, lg.shape, 1)
    i1 = jnp.argmax(lg, axis=-1).astype(jnp.int32)
    m1 = jnp.max(lg, axis=-1)
    masked = jnp.where(col == i1[:, None], -jnp.inf, lg)
    i2 = jnp.argmax(masked, axis=-1).astype(jnp.int32)
    m2 = jnp.max(masked, axis=-1)
    aa = jnp.exp(m2 - m1)
    w1 = 1.0 / (1.0 + aa)
    topi_ref[...] = jnp.stack([i1, i2], axis=-1)
    topw_ref[...] = jnp.stack([w1, 1.0 - w1], axis=-1)

    r = jax.lax.broadcasted_iota(jnp.int32, (SEG, SEG), 0)
    c = jax.lax.broadcasted_iota(jnp.int32, (SEG, SEG), 1)
    tri = (c < r).astype(jnp.float32)

    carry = carry_ref[...]
    ranks = []
    for oh in ((col == i1[:, None]).astype(jnp.float32),
               (col == i2[:, None]).astype(jnp.float32)):
        base = carry
        prefs = []
        for sgi in range(tg // SEG):
            seg = oh[sgi * SEG:(sgi + 1) * SEG, :]
            prefs.append(jnp.dot(tri, seg, preferred_element_type=jnp.float32) + base)
            base = base + jnp.sum(seg, axis=0, keepdims=True)
        rk = jnp.concatenate(prefs, axis=0)
        ranks.append(jnp.sum(rk * oh, axis=1))
        carry = base
    carry_ref[...] = carry
    cnt_ref[...] = carry
    rank_ref[...] = jnp.stack(ranks, axis=-1).astype(jnp.int32)


def _gating(x_flat, Wgate):
    t, h = x_flat.shape
    e = Wgate.shape[1]
    tg = 1024
    return pl.pallas_call(
        _gating_body,
        grid=(t // tg,),
        in_specs=[
            pl.BlockSpec((tg, h), lambda i: (i, 0)),
            pl.BlockSpec((h, e), lambda i: (0, 0)),
        ],
        out_specs=[
            pl.BlockSpec((tg, e), lambda i: (i, 0)),
            pl.BlockSpec((tg, TOPK), lambda i: (i, 0)),
            pl.BlockSpec((tg, TOPK), lambda i: (i, 0)),
            pl.BlockSpec((tg, TOPK), lambda i: (i, 0)),
            pl.BlockSpec((1, e), lambda i: (0, 0)),
        ],
        out_shape=[
            jax.ShapeDtypeStruct((t, e), jnp.float32),
            jax.ShapeDtypeStruct((t, TOPK), jnp.int32),
            jax.ShapeDtypeStruct((t, TOPK), jnp.float32),
            jax.ShapeDtypeStruct((t, TOPK), jnp.int32),
            jax.ShapeDtypeStruct((1, e), jnp.float32),
        ],
        scratch_shapes=[pltpu.VMEM((1, e), jnp.float32)],
    )(x_flat, Wgate)


def _dispatch(x_flat, slot0, slot1, p):
    t, h = x_flat.shape
    tok_per_w = t // NW
    ct = 64
    nchunks = tok_per_w // ct
    mesh = plsc.VectorSubcoreMesh(core_axis_name="c", subcore_axis_name="s")

    @functools.partial(
        pl.kernel,
        out_type=jax.ShapeDtypeStruct((p, h), jnp.float32),
        mesh=mesh,
        scratch_types=[
            pltpu.VMEM((ct,), jnp.int32),
            pltpu.VMEM((ct,), jnp.int32),
            pltpu.VMEM((ct, h), jnp.float32),
            pltpu.SemaphoreType.DMA,
        ],
    )
    def k(x_hbm, s0_hbm, s1_hbm, out_hbm, s0_v, s1_v, rows_v, sem):
        wid = lax.axis_index("s") * 2 + lax.axis_index("c")
        base = wid * tok_per_w

        def body(ci, carry):
            toff = base + ci * ct
            pltpu.sync_copy(x_hbm.at[pl.ds(toff, ct)], rows_v)
            pltpu.sync_copy(s0_hbm.at[pl.ds(toff, ct)], s0_v)
            pltpu.sync_copy(s1_hbm.at[pl.ds(toff, ct)], s1_v)
            cp0 = pltpu.async_copy(rows_v, out_hbm.at[s0_v], sem)
            cp1 = pltpu.async_copy(rows_v, out_hbm.at[s1_v], sem)
            cp0.wait()
            cp1.wait()
            return carry

        lax.fori_loop(0, nchunks, body, 0)

    return k(x_flat, slot0, slot1)


def _combine(ys, slot_flat, w_flat):
    p, h = ys.shape
    t = slot_flat.shape[0] // TOPK
    tok_per_w = t // NW
    ct = 16
    nchunks = tok_per_w // ct
    mesh = plsc.VectorSubcoreMesh(core_axis_name="c", subcore_axis_name="s")

    @functools.partial(
        pl.kernel,
        out_type=jax.ShapeDtypeStruct((t, h), jnp.float32),
        mesh=mesh,
        scratch_types=[
            pltpu.VMEM((tok_per_w * TOPK,), jnp.int32),
            pltpu.VMEM((tok_per_w * TOPK,), jnp.float32),
            pltpu.VMEM((2, TOPK * ct, h), jnp.float32),
            pltpu.VMEM((ct, h), jnp.float32),
            pltpu.SemaphoreType.DMA,
            pltpu.SemaphoreType.DMA,
        ],
    )
    def k(ys_hbm, pos_hbm, w_hbm, out_hbm, idx_v, w_v, rows_v, out_v, sem0, sem1):
        wid = lax.axis_index("s") * 2 + lax.axis_index("c")
        tbase = wid * tok_per_w
        sems = [sem0, sem1]
        pltpu.sync_copy(pos_hbm.at[pl.ds(TOPK * tbase, TOPK * tok_per_w)], idx_v)
        pltpu.sync_copy(w_hbm.at[pl.ds(TOPK * tbase, TOPK * tok_per_w)], w_v)

        def gather(g, b):
            return pltpu.async_copy(
                ys_hbm.at[idx_v.at[pl.ds(g * TOPK * ct, TOPK * ct)]],
                rows_v.at[b],
                sems[b],
            )

        gather(0, 0)
        for g in range(nchunks):
            b = g % 2
            if g + 1 < nchunks:
                gather(g + 1, 1 - b)
            pltpu.make_async_copy(
                ys_hbm.at[idx_v.at[pl.ds(g * TOPK * ct, TOPK * ct)]],
                rows_v.at[b],
                sems[b],
            ).wait()
            wva = w_v[pl.ds(g * TOPK * ct, LANES)]
            wvb = w_v[pl.ds(g * TOPK * ct + LANES, LANES)]
            for i in range(ct):
                wv = wva if 2 * i < LANES else wvb
                w0 = wv[(2 * i) % LANES]
                w1 = wv[(2 * i + 1) % LANES]

                def hbody(j, carry3, b=b, i=i, w0=w0, w1=w1):
                    sl = pl.ds(j * LANES, LANES)
                    out_v[i, sl] = (
                        w0 * rows_v[b, 2 * i, sl] + w1 * rows_v[b, 2 * i + 1, sl]
                    )
                    return carry3

                lax.fori_loop(0, h // LANES, hbody, 0)

            pltpu.sync_copy(out_v, out_hbm.at[pl.ds(tbase + g * ct, ct)])

    return k(ys, slot_flat, w_flat)


def _cast_body(wg_ref, wu_ref, wd_ref, og_ref, ou_ref, od_ref):
    og_ref[...] = wg_ref[...].astype(jnp.bfloat16)
    ou_ref[...] = wu_ref[...].astype(jnp.bfloat16)
    od_ref[...] = wd_ref[...].astype(jnp.bfloat16)


def _cast_weights(Wg, Wu, Wd):
    e, h, f = Wg.shape
    nc = 2
    fc = f // nc
    return pl.pallas_call(
        _cast_body,
        grid=(e, nc),
        in_specs=[
            pl.BlockSpec((1, h, fc), lambda i, j: (i, 0, j)),
            pl.BlockSpec((1, h, fc), lambda i, j: (i, 0, j)),
            pl.BlockSpec((1, fc, h), lambda i, j: (i, j, 0)),
        ],
        out_specs=[
            pl.BlockSpec((1, h, fc), lambda i, j: (i, 0, j)),
            pl.BlockSpec((1, h, fc), lambda i, j: (i, 0, j)),
            pl.BlockSpec((1, fc, h), lambda i, j: (i, j, 0)),
        ],
        out_shape=[
            jax.ShapeDtypeStruct((e, h, f), jnp.bfloat16),
            jax.ShapeDtypeStruct((e, h, f), jnp.bfloat16),
            jax.ShapeDtypeStruct((e, f, h), jnp.bfloat16),
        ],
    )(Wg, Wu, Wd)


def _ffn_body(be_ref, xs_ref, wg_ref, wu_ref, wd_ref, ys_ref):
    del be_ref
    xb = xs_ref[...].astype(jnp.bfloat16)
    g = jnp.dot(xb, wg_ref[0], preferred_element_type=jnp.float32)
    u = jnp.dot(xb, wu_ref[0], preferred_element_type=jnp.float32)
    h1 = (g * jax.nn.sigmoid(g) * u).astype(jnp.bfloat16)
    ys_ref[...] = jnp.dot(h1, wd_ref[0], preferred_element_type=jnp.float32)


def _grouped_ffn(xs, Wg, Wu, Wd, block_expert):
    p, h = xs.shape
    _, _, f = Wg.shape
    nb = p // BT
    grid_spec = pltpu.PrefetchScalarGridSpec(
        num_scalar_prefetch=1,
        grid=(nb,),
        in_specs=[
            pl.BlockSpec((BT, h), lambda i, be: (i, 0)),
            pl.BlockSpec((1, h, f), lambda i, be: (be[i], 0, 0)),
            pl.BlockSpec((1, h, f), lambda i, be: (be[i], 0, 0)),
            pl.BlockSpec((1, f, h), lambda i, be: (be[i], 0, 0)),
        ],
        out_specs=pl.BlockSpec((BT, h), lambda i, be: (i, 0)),
    )
    return pl.pallas_call(
        _ffn_body,
        grid_spec=grid_spec,
        out_shape=jax.ShapeDtypeStruct((p, h), jnp.float32),
    )(block_expert, xs, Wg, Wu, Wd)


@jax.jit
def kernel(x, Wgate, Wg, Wu, Wd):
    b, s, h = x.shape
    e = Wgate.shape[1]
    t = b * s
    a = t * TOPK
    p = a + e * BT
    nb = p // BT

    x_flat = x.reshape(t, h)
    logits, topi, topw, rank2, cnt_f = _gating(x_flat, Wgate)

    cnt = cnt_f[0].astype(jnp.int32)
    cnt_pad = ((cnt + BT - 1) // BT) * BT
    ends = jnp.cumsum(cnt_pad)
    aligned_off = ends - cnt_pad
    slot2 = aligned_off[topi] + rank2
    slot = slot2.reshape(-1)
    blockstart = jnp.arange(nb, dtype=jnp.int32) * BT
    block_expert = jnp.minimum(
        jnp.sum((blockstart[:, None] >= ends[None, :]).astype(jnp.int32), axis=1),
        e - 1,
    ).astype(jnp.int32)

    wg16, wu16, wd16 = _cast_weights(Wg, Wu, Wd)
    xs = _dispatch(x_flat, slot2[:, 0], slot2[:, 1], p)

    ys = _grouped_ffn(xs, wg16, wu16, wd16, block_expert)

    out = _combine(ys, slot, topw.reshape(-1))

    return out.reshape(b, s, h), logits
